# baseline (speedup 1.0000x reference)
import functools

import jax
import jax.numpy as jnp
from jax import lax
from jax.experimental import pallas as pl
from jax.experimental.pallas import tpu as pltpu
from jax.experimental.pallas import tpu_sc as plsc

EMB = 64
HID = 64
OUT = 32
NC = 2
CHUNK = 2048


def _sc_gather(emb, idx_flat):
    m = idx_flat.shape[0]
    per_core = m // NC
    mesh = plsc.ScalarSubcoreMesh(axis_name="c")

    @functools.partial(
        pl.kernel,
        mesh=mesh,
        out_type=jax.ShapeDtypeStruct((m, EMB), jnp.float32),
        scratch_types=[
            pltpu.SMEM((CHUNK,), jnp.int32),
            pltpu.SemaphoreType.DMA,
            pltpu.SemaphoreType.DMA,
        ],
    )
    def gather_kernel(table_hbm, idx_hbm, out_hbm, i_s, sem_i, sem):
        cid = lax.axis_index("c")
        base = cid * per_core

        @pl.loop(0, per_core, step=CHUNK)
        def _(off):
            pltpu.async_copy(
                idx_hbm.at[pl.ds(base + off, CHUNK)], i_s, sem_i
            ).wait()

            @pl.loop(0, CHUNK)
            def _(j):
                a = i_s[j]
                pltpu.async_copy(
                    table_hbm.at[pl.ds(a, 1)],
                    out_hbm.at[pl.ds(base + off + j, 1)],
                    sem,
                )

        pltpu.make_async_copy(
            table_hbm.at[pl.ds(0, per_core)],
            out_hbm.at[pl.ds(base, per_core)],
            sem,
        ).wait()

    return gather_kernel(emb, idx_flat)


def _mlp(g, w1aT, w1bT, b1, w2T, b2, w3T, b3):
    n = g.shape[0] // 2
    blk = 2048
    nb = n // blk

    def body(x1_ref, x2_ref, w1a_ref, w1b_ref, b1_ref, w2_ref, b2_ref,
             w3_ref, b3_ref, o_ref):
        a = jnp.dot(x1_ref[...], w1a_ref[...], preferred_element_type=jnp.float32)
        a = a + jnp.dot(x2_ref[...], w1b_ref[...], preferred_element_type=jnp.float32)
        a = jnp.maximum(a + b1_ref[...], 0.0)
        a = jnp.dot(a, w2_ref[...], preferred_element_type=jnp.float32) + b2_ref[...]
        a = jnp.maximum(a, 0.0)
        o_ref[...] = jnp.dot(a, w3_ref[...], preferred_element_type=jnp.float32) + b3_ref[...]

    full = lambda shape: pl.BlockSpec(shape, lambda i: (0, 0))
    return pl.pallas_call(
        body,
        grid=(nb,),
        in_specs=[
            pl.BlockSpec((blk, EMB), lambda i: (i, 0)),
            pl.BlockSpec((blk, EMB), lambda i: (i + nb, 0)),
            full((EMB, HID)),
            full((EMB, HID)),
            full((1, HID)),
            full((HID, HID)),
            full((1, HID)),
            full((HID, OUT)),
            full((1, OUT)),
        ],
        out_specs=pl.BlockSpec((blk, OUT), lambda i: (i, 0)),
        out_shape=jax.ShapeDtypeStruct((n, OUT), jnp.float32),
    )(g, g, w1aT, w1bT, b1, w2T, b2, w3T, b3)


def kernel(x, emb, w1, b1, w2, b2, w3, b3):
    xi = x.astype(jnp.int32)
    idx_flat = xi.T.reshape(-1)
    g = _sc_gather(emb, idx_flat)
    return _mlp(
        g,
        w1[:, :EMB].T,
        w1[:, EMB:].T,
        b1.reshape(1, HID),
        w2.T,
        b2.reshape(1, HID),
        w3.T,
        b3.reshape(1, OUT),
    )


# speedup vs baseline: 2.2566x; 2.2566x over previous
import functools

import jax
import jax.numpy as jnp
from jax import lax
from jax.experimental import pallas as pl
from jax.experimental.pallas import tpu as pltpu
from jax.experimental.pallas import tpu_sc as plsc

EMB = 64
HID = 64
OUT = 32
NC = 2
NS = 16
NW = NC * NS
LANES = 16


def _sc_gather(emb, idx_flat):
    m = idx_flat.shape[0]
    per_w = m // NW
    mesh = plsc.VectorSubcoreMesh(core_axis_name="c", subcore_axis_name="s")

    @functools.partial(
        pl.kernel,
        mesh=mesh,
        out_type=jax.ShapeDtypeStruct((m, EMB), jnp.float32),
        scratch_types=[
            pltpu.VMEM((per_w,), jnp.int32),
            pltpu.VMEM((512, EMB), jnp.float32),
            pltpu.SemaphoreType.DMA,
        ],
    )
    def gather_kernel(table_hbm, idx_hbm, out_hbm, i_v, rows_v, sem):
        wid = lax.axis_index("s") * NC + lax.axis_index("c")
        base = wid * per_w
        pltpu.sync_copy(idx_hbm.at[pl.ds(base, per_w)], i_v)

        @pl.loop(0, per_w, step=512)
        def _(off):
            @pl.loop(0, 512, step=LANES)
            def _(j0):
                vec = i_v[pl.ds(off + j0, LANES)]
                for t in range(LANES):
                    a = vec[t]
                    pltpu.async_copy(
                        table_hbm.at[pl.ds(a, 1)],
                        rows_v.at[pl.ds(j0 + t, 1)],
                        sem,
                    )

            pltpu.make_async_copy(
                table_hbm.at[pl.ds(0, 512)], rows_v, sem
            ).wait()
            pltpu.sync_copy(rows_v, out_hbm.at[pl.ds(base + off, 512)])

    return gather_kernel(emb, idx_flat)


def _mlp(g, w1aT, w1bT, b1, w2T, b2, w3T, b3):
    n = g.shape[0] // 2
    blk = 2048
    nb = n // blk

    def body(x1_ref, x2_ref, w1a_ref, w1b_ref, b1_ref, w2_ref, b2_ref,
             w3_ref, b3_ref, o_ref):
        a = jnp.dot(x1_ref[...], w1a_ref[...], preferred_element_type=jnp.float32)
        a = a + jnp.dot(x2_ref[...], w1b_ref[...], preferred_element_type=jnp.float32)
        a = jnp.maximum(a + b1_ref[...], 0.0)
        a = jnp.dot(a, w2_ref[...], preferred_element_type=jnp.float32) + b2_ref[...]
        a = jnp.maximum(a, 0.0)
        o_ref[...] = jnp.dot(a, w3_ref[...], preferred_element_type=jnp.float32) + b3_ref[...]

    full = lambda shape: pl.BlockSpec(shape, lambda i: (0, 0))
    return pl.pallas_call(
        body,
        grid=(nb,),
        in_specs=[
            pl.BlockSpec((blk, EMB), lambda i: (i, 0)),
            pl.BlockSpec((blk, EMB), lambda i: (i + nb, 0)),
            full((EMB, HID)),
            full((EMB, HID)),
            full((1, HID)),
            full((HID, HID)),
            full((1, HID)),
            full((HID, OUT)),
            full((1, OUT)),
        ],
        out_specs=pl.BlockSpec((blk, OUT), lambda i: (i, 0)),
        out_shape=jax.ShapeDtypeStruct((n, OUT), jnp.float32),
    )(g, g, w1aT, w1bT, b1, w2T, b2, w3T, b3)


def kernel(x, emb, w1, b1, w2, b2, w3, b3):
    xi = x.astype(jnp.int32)
    idx_flat = xi.T.reshape(-1)
    g = _sc_gather(emb, idx_flat)
    return _mlp(
        g,
        w1[:, :EMB].T,
        w1[:, EMB:].T,
        b1.reshape(1, HID),
        w2.T,
        b2.reshape(1, HID),
        w3.T,
        b3.reshape(1, OUT),
    )
